# baseline (speedup 1.0000x reference)
import functools

import numpy as np

import jax
import jax.numpy as jnp
from jax import lax
from jax.experimental import pallas as pl
from jax.experimental.pallas import tpu as pltpu
from jax.experimental.pallas import tpu_sc as plsc

N_NODES = 10000
N_EDGES = 320000

NC, NS = 2, 16
NW = NC * NS
CHUNK = 128
EDGES_PER_TILE = 10240
NCHUNKS = EDGES_PER_TILE // CHUNK
E_PAD = NW * EDGES_PER_TILE
N_ACC = 10112
ZROWS = N_ACC // NS
WB_ROWS = 624

_PAD_N = E_PAD - N_EDGES
_PAD_SRC = np.arange(_PAD_N, dtype=np.int32) % N_NODES
_PAD_DST = (N_NODES + np.arange(_PAD_N, dtype=np.int32) % (N_ACC - N_NODES)
            ).astype(np.int32)


def _write_back(acc, out, c, s):
  pltpu.sync_copy(acc.at[pl.ds(s * WB_ROWS, WB_ROWS)],
                  out.at[c, pl.ds(s * WB_ROWS, WB_ROWS)])

  @pl.when(s == NS - 1)
  def _tail():
    base = NS * WB_ROWS
    pltpu.sync_copy(acc.at[pl.ds(base, N_NODES - base)],
                    out.at[c, pl.ds(base, N_NODES - base)])


def _make_sc_cnt():
  mesh = plsc.VectorSubcoreMesh(core_axis_name="c", subcore_axis_name="s")

  @functools.partial(
      pl.kernel,
      out_type=jax.ShapeDtypeStruct((NC, N_NODES, 16), jnp.float32),
      mesh=mesh,
      scratch_types=[
          pltpu.VMEM((NCHUNKS, CHUNK), jnp.int32),
          pltpu.VMEM((CHUNK, 16), jnp.float32),
          pltpu.VMEM_SHARED((N_ACC, 16), jnp.float32),
          pltpu.SemaphoreType.DMA,
      ],
      compiler_params=pltpu.CompilerParams(use_tc_tiling_on_sc=False),
  )
  def cnt(dstp, zrows, out, dst_v, ones_v, acc, sem):
    c = lax.axis_index("c")
    s = lax.axis_index("s")
    wid = c * NS + s

    pltpu.sync_copy(zrows, acc.at[pl.ds(s * ZROWS, ZROWS)])
    pltpu.sync_copy(dstp.at[wid], dst_v)

    def fill_ones(r, carry):
      ones_v[r] = jnp.ones((16,), jnp.float32)
      return carry

    lax.fori_loop(0, CHUNK, fill_ones, 0)
    plsc.subcore_barrier()

    K = 16

    def outer(io, carry):
      base = io * K
      for k in range(K):
        pltpu.async_copy(ones_v, acc.at[dst_v.at[base + k]], sem, add=True)
      for k in range(K):
        pltpu.make_async_copy(ones_v, acc.at[dst_v.at[base + k]], sem).wait()
      return carry

    lax.fori_loop(0, NCHUNKS // K, outer, 0)
    plsc.subcore_barrier()
    _write_back(acc, out, c, s)

  return cnt


def _make_sc_agg(D, nbuf, stages, with_counts, tc_tiling):
  mesh = plsc.VectorSubcoreMesh(core_axis_name="c", subcore_axis_name="s")
  assert sum(stages) == NCHUNKS
  smax = max(stages)

  out_type = [jax.ShapeDtypeStruct((NC, N_NODES, D), jnp.float32)]
  scratch = [
      pltpu.VMEM((smax, CHUNK), jnp.int32),
      pltpu.VMEM((smax, CHUNK), jnp.int32),
      [pltpu.VMEM((CHUNK, D), jnp.float32) for _ in range(nbuf)],
      pltpu.VMEM_SHARED((N_ACC, D), jnp.float32),
      [pltpu.SemaphoreType.DMA] * nbuf,
  ]
  if with_counts:
    out_type.append(jax.ShapeDtypeStruct((NC, N_NODES, 16), jnp.float32))
    scratch += [
        pltpu.VMEM((CHUNK, 16), jnp.float32),
        pltpu.VMEM_SHARED((N_ACC, 16), jnp.float32),
        pltpu.SemaphoreType.DMA,
    ]

  @functools.partial(
      pl.kernel,
      out_type=out_type,
      mesh=mesh,
      scratch_types=scratch,
      compiler_params=pltpu.CompilerParams(use_tc_tiling_on_sc=tc_tiling),
  )
  def agg(table, srcp, dstp, zrows, *rest):
    if with_counts:
      (zrows16, out, out_c, src_v, dst_v, rows_v, acc, sems,
       ones_v, acc_c, sem_c) = rest
    else:
      out, src_v, dst_v, rows_v, acc, sems = rest
    c = lax.axis_index("c")
    s = lax.axis_index("s")
    wid = c * NS + s

    pltpu.sync_copy(zrows, acc.at[pl.ds(s * ZROWS, ZROWS)])
    if with_counts:
      pltpu.sync_copy(zrows16, acc_c.at[pl.ds(s * ZROWS, ZROWS)])

      def fill_ones(r, carry):
        ones_v[r] = jnp.ones((16,), jnp.float32)
        return carry

      lax.fori_loop(0, CHUNK, fill_ones, 0)

    def load_idx(off, size):
      pltpu.sync_copy(srcp.at[wid, pl.ds(off, size)],
                      src_v.at[pl.ds(0, size)])
      pltpu.sync_copy(dstp.at[wid, pl.ds(off, size)],
                      dst_v.at[pl.ds(0, size)])

    load_idx(0, stages[0])
    plsc.subcore_barrier()

    def run_chunks(size):
      for b in range(nbuf):
        pltpu.async_copy(table.at[src_v.at[b]], rows_v[b], sems[b])

      def outer(io, carry):
        for b in range(nbuf):
          i = io * nbuf + b
          pltpu.make_async_copy(table.at[src_v.at[i]], rows_v[b],
                                sems[b]).wait()
          pltpu.sync_copy(rows_v[b], acc.at[dst_v.at[i]], add=True)
          if with_counts:
            pltpu.async_copy(ones_v, acc_c.at[dst_v.at[i]], sem_c,
                             add=True)

          @pl.when(i + nbuf < size)
          def _refill():
            pltpu.async_copy(table.at[src_v.at[i + nbuf]], rows_v[b],
                             sems[b])
        return carry

      lax.fori_loop(0, size // nbuf, outer, 0)
      if with_counts:
        def drain(io, carry):
          pltpu.make_async_copy(ones_v, acc_c.at[dst_v.at[0]],
                                sem_c).wait()
          return carry
        lax.fori_loop(0, size, drain, 0)

    run_chunks(stages[0])
    off = stages[0]
    for sz in stages[1:]:
      load_idx(off, sz)
      run_chunks(sz)
      off += sz

    plsc.subcore_barrier()
    _write_back(acc, out, c, s)
    if with_counts:
      _write_back(acc_c, out_c, c, s)

  return agg


_sc_cnt = _make_sc_cnt()
_sc_agg_128 = _make_sc_agg(128, nbuf=2, stages=(40, 40), with_counts=False,
                           tc_tiling=True)
_sc_agg_16 = _make_sc_agg(16, nbuf=8, stages=(80,), with_counts=False,
                          tc_tiling=False)


_TC_BLK = 5000
_GRID = N_NODES // _TC_BLK


def _tc1_body(s1_ref, cnt_ref, x_ref, w1r_ref, b1_ref, w1t_ref, w2r_ref,
              w2t_ref, b2_ref, p2_ref, r2_ref, ic_ref):
  ic = 1.0 / jnp.maximum(cnt_ref[0, :, :8] + cnt_ref[1, :, :8], 1.0)
  agg = (s1_ref[0] + s1_ref[1]) * ic[:, :1]
  h1 = jnp.maximum(
      jnp.dot(agg, w1r_ref[...], preferred_element_type=jnp.float32)
      + b1_ref[...]
      + jnp.dot(x_ref[...], w1t_ref[...], preferred_element_type=jnp.float32),
      0.0)
  p2_ref[...] = jnp.dot(h1, w2r_ref[...], preferred_element_type=jnp.float32)
  r2_ref[...] = (
      jnp.dot(h1, w2t_ref[...], preferred_element_type=jnp.float32)
      + b2_ref[...])
  ic_ref[...] = ic


def _tc2_body(s2_ref, r2_ref, ic_ref, wmr_ref, wlr_ref, wmt_ref, wlt_ref,
              bm_ref, bl_ref, p3_ref, r3_ref):
  h2 = jnp.maximum(
      (s2_ref[0] + s2_ref[1]) * ic_ref[:, :1] + r2_ref[...], 0.0)
  w3r = jnp.concatenate(
      [wmr_ref[...], wlr_ref[...],
       jnp.zeros((128, 12), jnp.float32)], axis=1)
  w3t = jnp.concatenate(
      [wmt_ref[...], wlt_ref[...],
       jnp.zeros((128, 12), jnp.float32)], axis=1)
  b3 = jnp.concatenate(
      [bm_ref[...], bl_ref[...], jnp.zeros((1, 12), jnp.float32)], axis=1)
  p3_ref[...] = jnp.dot(h2, w3r, preferred_element_type=jnp.float32)
  r3_ref[...] = jnp.dot(h2, w3t, preferred_element_type=jnp.float32) + b3


def _tc3_body(s3_ref, r3_ref, ic_ref, out_ref):
  out_ref[...] = (s3_ref[0] + s3_ref[1]) * ic_ref[:, :1] + r3_ref[...]


def _row_blk(shape_tail):
  return pl.BlockSpec((_TC_BLK,) + shape_tail,
                      lambda i: (i,) + (0,) * len(shape_tail))


def _part_blk(d):
  return pl.BlockSpec((NC, _TC_BLK, d), lambda i: (0, i, 0))


def _full_blk(shape):
  return pl.BlockSpec(shape, lambda i: (0,) * len(shape))


def kernel(x, W1_rel, b1, W1_root, W2_rel, b2, W2_root, Wmu_rel, bmu,
           Wmu_root, Wls_rel, bls, Wls_root, edge_index):
  src = edge_index[0].astype(jnp.int32)
  dst = edge_index[1].astype(jnp.int32)
  srcp = jnp.concatenate([src, _PAD_SRC]).reshape(NW, NCHUNKS, CHUNK)
  dstp = jnp.concatenate([dst, _PAD_DST]).reshape(NW, NCHUNKS, CHUNK)

  z128 = jnp.zeros((ZROWS, 128), jnp.float32)
  z16 = jnp.zeros((ZROWS, 16), jnp.float32)

  cnt = _sc_cnt(dstp, z16)
  (s1,) = _sc_agg_128(x, srcp, dstp, z128)

  p2, r2, ic = pl.pallas_call(
      _tc1_body,
      grid=(_GRID,),
      in_specs=[
          _part_blk(128),
          _part_blk(16),
          _row_blk((128,)),
          _full_blk((128, 256)),
          _full_blk((1, 256)),
          _full_blk((128, 256)),
          _full_blk((256, 128)),
          _full_blk((256, 128)),
          _full_blk((1, 128)),
      ],
      out_specs=[_row_blk((128,)), _row_blk((128,)), _row_blk((8,))],
      out_shape=[
          jax.ShapeDtypeStruct((N_NODES, 128), jnp.float32),
          jax.ShapeDtypeStruct((N_NODES, 128), jnp.float32),
          jax.ShapeDtypeStruct((N_NODES, 8), jnp.float32),
      ],
  )(s1, cnt, x, W1_rel, b1.reshape(1, 256), W1_root, W2_rel, W2_root,
    b2.reshape(1, 128))

  (s2,) = _sc_agg_128(p2, srcp, dstp, z128)

  p3, r3 = pl.pallas_call(
      _tc2_body,
      grid=(_GRID,),
      in_specs=[
          _part_blk(128),
          _row_blk((128,)),
          _row_blk((8,)),
          _full_blk((128, 2)),
          _full_blk((128, 2)),
          _full_blk((128, 2)),
          _full_blk((128, 2)),
          _full_blk((1, 2)),
          _full_blk((1, 2)),
      ],
      out_specs=[_row_blk((16,)), _row_blk((16,))],
      out_shape=[
          jax.ShapeDtypeStruct((N_NODES, 16), jnp.float32),
          jax.ShapeDtypeStruct((N_NODES, 16), jnp.float32),
      ],
  )(s2, r2, ic, Wmu_rel, Wls_rel, Wmu_root, Wls_root,
    bmu.reshape(1, 2), bls.reshape(1, 2))

  (s3,) = _sc_agg_16(p3, srcp, dstp, z16)

  out = pl.pallas_call(
      _tc3_body,
      grid=(_GRID,),
      in_specs=[_part_blk(16), _row_blk((16,)), _row_blk((8,))],
      out_specs=_row_blk((16,)),
      out_shape=jax.ShapeDtypeStruct((N_NODES, 16), jnp.float32),
  )(s3, r3, ic)

  return out[:, 0:2], out[:, 2:4]


# speedup vs baseline: 1.0176x; 1.0176x over previous
import functools

import numpy as np

import jax
import jax.numpy as jnp
from jax import lax
from jax.experimental import pallas as pl
from jax.experimental.pallas import tpu as pltpu
from jax.experimental.pallas import tpu_sc as plsc

N_NODES = 10000
N_EDGES = 320000

NC, NS = 2, 16
NW = NC * NS
CHUNK = 128
EDGES_PER_TILE = 10240
NCHUNKS = EDGES_PER_TILE // CHUNK
E_PAD = NW * EDGES_PER_TILE
N_ACC = 10112
ZROWS = N_ACC // NS
WB_ROWS = 624

_PAD_N = E_PAD - N_EDGES
_PAD_SRC = np.arange(_PAD_N, dtype=np.int32) % N_NODES
_PAD_DST = (N_NODES + np.arange(_PAD_N, dtype=np.int32) % (N_ACC - N_NODES)
            ).astype(np.int32)


def _write_back(acc, out, c, s):
  pltpu.sync_copy(acc.at[pl.ds(s * WB_ROWS, WB_ROWS)],
                  out.at[c, pl.ds(s * WB_ROWS, WB_ROWS)])

  @pl.when(s == NS - 1)
  def _tail():
    base = NS * WB_ROWS
    pltpu.sync_copy(acc.at[pl.ds(base, N_NODES - base)],
                    out.at[c, pl.ds(base, N_NODES - base)])


def _make_sc_cnt():
  mesh = plsc.VectorSubcoreMesh(core_axis_name="c", subcore_axis_name="s")

  @functools.partial(
      pl.kernel,
      out_type=jax.ShapeDtypeStruct((NC, N_NODES, 16), jnp.float32),
      mesh=mesh,
      scratch_types=[
          pltpu.VMEM((NCHUNKS, CHUNK), jnp.int32),
          pltpu.VMEM((CHUNK, 16), jnp.float32),
          pltpu.VMEM_SHARED((N_ACC, 16), jnp.float32),
          pltpu.SemaphoreType.DMA,
      ],
      compiler_params=pltpu.CompilerParams(use_tc_tiling_on_sc=False),
  )
  def cnt(dstp, zrows, out, dst_v, ones_v, acc, sem):
    c = lax.axis_index("c")
    s = lax.axis_index("s")
    wid = c * NS + s

    pltpu.sync_copy(zrows, acc.at[pl.ds(s * ZROWS, ZROWS)])
    pltpu.sync_copy(dstp.at[wid], dst_v)

    def fill_ones(r, carry):
      ones_v[r] = jnp.ones((16,), jnp.float32)
      return carry

    lax.fori_loop(0, CHUNK, fill_ones, 0)
    plsc.subcore_barrier()

    K = 16

    def outer(io, carry):
      base = io * K
      for k in range(K):
        pltpu.async_copy(ones_v, acc.at[dst_v.at[base + k]], sem, add=True)
      for k in range(K):
        pltpu.make_async_copy(ones_v, acc.at[dst_v.at[base + k]], sem).wait()
      return carry

    lax.fori_loop(0, NCHUNKS // K, outer, 0)
    plsc.subcore_barrier()
    _write_back(acc, out, c, s)

  return cnt


def _make_sc_agg(D, nbuf, stages, with_counts, tc_tiling):
  mesh = plsc.VectorSubcoreMesh(core_axis_name="c", subcore_axis_name="s")
  assert sum(stages) == NCHUNKS
  smax = max(stages)

  out_type = [jax.ShapeDtypeStruct((NC, N_NODES, D), jnp.float32)]
  scratch = [
      pltpu.VMEM((smax, CHUNK), jnp.int32),
      pltpu.VMEM((smax, CHUNK), jnp.int32),
      [pltpu.VMEM((CHUNK, D), jnp.float32) for _ in range(nbuf)],
      pltpu.VMEM_SHARED((N_ACC, D), jnp.float32),
      [pltpu.SemaphoreType.DMA] * nbuf,
  ]
  if with_counts:
    out_type.append(jax.ShapeDtypeStruct((NC, N_NODES, 16), jnp.float32))
    scratch += [
        pltpu.VMEM((CHUNK, 16), jnp.float32),
        pltpu.VMEM_SHARED((N_ACC, 16), jnp.float32),
        pltpu.SemaphoreType.DMA,
    ]

  @functools.partial(
      pl.kernel,
      out_type=out_type,
      mesh=mesh,
      scratch_types=scratch,
      compiler_params=pltpu.CompilerParams(use_tc_tiling_on_sc=False),
  )
  def agg(table, srcp, dstp, zrows, *rest):
    if with_counts:
      (zrows16, out, out_c, src_v, dst_v, rows_v, acc, sems,
       ones_v, acc_c, sem_c) = rest
    else:
      out, src_v, dst_v, rows_v, acc, sems = rest
    c = lax.axis_index("c")
    s = lax.axis_index("s")
    wid = c * NS + s

    pltpu.sync_copy(zrows, acc.at[pl.ds(s * ZROWS, ZROWS)])
    if with_counts:
      pltpu.sync_copy(zrows16, acc_c.at[pl.ds(s * ZROWS, ZROWS)])

      def fill_ones(r, carry):
        ones_v[r] = jnp.ones((16,), jnp.float32)
        return carry

      lax.fori_loop(0, CHUNK, fill_ones, 0)

    def load_idx(off, size):
      pltpu.sync_copy(srcp.at[wid, pl.ds(off, size)],
                      src_v.at[pl.ds(0, size)])
      pltpu.sync_copy(dstp.at[wid, pl.ds(off, size)],
                      dst_v.at[pl.ds(0, size)])

    load_idx(0, stages[0])
    plsc.subcore_barrier()

    def run_chunks(size):
      for b in range(nbuf):
        pltpu.async_copy(table.at[src_v.at[b]], rows_v[b], sems[b])

      def outer(io, carry):
        for b in range(nbuf):
          i = io * nbuf + b
          pltpu.make_async_copy(table.at[src_v.at[i]], rows_v[b],
                                sems[b]).wait()
          pltpu.sync_copy(rows_v[b], acc.at[dst_v.at[i]], add=True)
          if with_counts:
            pltpu.async_copy(ones_v, acc_c.at[dst_v.at[i]], sem_c,
                             add=True)

          @pl.when(i + nbuf < size)
          def _refill():
            pltpu.async_copy(table.at[src_v.at[i + nbuf]], rows_v[b],
                             sems[b])
        return carry

      lax.fori_loop(0, size // nbuf, outer, 0)
      if with_counts:
        def drain(io, carry):
          pltpu.make_async_copy(ones_v, acc_c.at[dst_v.at[0]],
                                sem_c).wait()
          return carry
        lax.fori_loop(0, size, drain, 0)

    run_chunks(stages[0])
    off = stages[0]
    for sz in stages[1:]:
      load_idx(off, sz)
      run_chunks(sz)
      off += sz

    plsc.subcore_barrier()
    _write_back(acc, out, c, s)
    if with_counts:
      _write_back(acc_c, out_c, c, s)

  return agg


_sc_cnt = _make_sc_cnt()
_sc_agg_128 = _make_sc_agg(128, nbuf=2, stages=(40, 40), with_counts=False,
                           tc_tiling=True)
_sc_agg_16 = _make_sc_agg(16, nbuf=8, stages=(80,), with_counts=False,
                          tc_tiling=False)


_TC_BLK = 5000
_GRID = N_NODES // _TC_BLK


def _tc1_body(s1_ref, ic_ref, x_ref, w1r_ref, b1_ref, w1t_ref, w2r_ref,
              w2t_ref, b2_ref, p2_ref, r2_ref):
  ic = ic_ref[...]
  agg = (s1_ref[0] + s1_ref[1]) * ic[:, :1]
  h1 = jnp.maximum(
      jnp.dot(agg, w1r_ref[...], preferred_element_type=jnp.float32)
      + b1_ref[...]
      + jnp.dot(x_ref[...], w1t_ref[...], preferred_element_type=jnp.float32),
      0.0)
  p2_ref[...] = jnp.dot(h1, w2r_ref[...], preferred_element_type=jnp.float32)
  r2_ref[...] = (
      jnp.dot(h1, w2t_ref[...], preferred_element_type=jnp.float32)
      + b2_ref[...])


def _tc2_body(s2_ref, r2_ref, ic_ref, wmr_ref, wlr_ref, wmt_ref, wlt_ref,
              bm_ref, bl_ref, p3_ref, r3_ref):
  h2 = jnp.maximum(
      (s2_ref[0] + s2_ref[1]) * ic_ref[:, :1] + r2_ref[...], 0.0)
  w3r = jnp.concatenate(
      [wmr_ref[...], wlr_ref[...],
       jnp.zeros((128, 12), jnp.float32)], axis=1)
  w3t = jnp.concatenate(
      [wmt_ref[...], wlt_ref[...],
       jnp.zeros((128, 12), jnp.float32)], axis=1)
  b3 = jnp.concatenate(
      [bm_ref[...], bl_ref[...], jnp.zeros((1, 12), jnp.float32)], axis=1)
  p3_ref[...] = jnp.dot(h2, w3r, preferred_element_type=jnp.float32)
  r3_ref[...] = jnp.dot(h2, w3t, preferred_element_type=jnp.float32) + b3


def _row_blk(shape_tail):
  return pl.BlockSpec((_TC_BLK,) + shape_tail,
                      lambda i: (i,) + (0,) * len(shape_tail))


def _part_blk(d):
  return pl.BlockSpec((NC, _TC_BLK, d), lambda i: (0, i, 0))


def _full_blk(shape):
  return pl.BlockSpec(shape, lambda i: (0,) * len(shape))


def kernel(x, W1_rel, b1, W1_root, W2_rel, b2, W2_root, Wmu_rel, bmu,
           Wmu_root, Wls_rel, bls, Wls_root, edge_index):
  src = edge_index[0].astype(jnp.int32)
  dst = edge_index[1].astype(jnp.int32)
  srcp = jnp.concatenate([src, _PAD_SRC]).reshape(NW, NCHUNKS, CHUNK)
  dstp = jnp.concatenate([dst, _PAD_DST]).reshape(NW, NCHUNKS, CHUNK)

  z128 = jnp.zeros((ZROWS, 128), jnp.float32)
  z16 = jnp.zeros((ZROWS, 16), jnp.float32)

  cnt = _sc_cnt(dstp, z16)
  ic = 1.0 / jnp.maximum(cnt[0, :, :8] + cnt[1, :, :8], 1.0)
  (s1,) = _sc_agg_128(x, srcp, dstp, z128)

  p2, r2 = pl.pallas_call(
      _tc1_body,
      grid=(_GRID,),
      in_specs=[
          _part_blk(128),
          _row_blk((8,)),
          _row_blk((128,)),
          _full_blk((128, 256)),
          _full_blk((1, 256)),
          _full_blk((128, 256)),
          _full_blk((256, 128)),
          _full_blk((256, 128)),
          _full_blk((1, 128)),
      ],
      out_specs=[_row_blk((128,)), _row_blk((128,))],
      out_shape=[
          jax.ShapeDtypeStruct((N_NODES, 128), jnp.float32),
          jax.ShapeDtypeStruct((N_NODES, 128), jnp.float32),
      ],
  )(s1, ic, x, W1_rel, b1.reshape(1, 256), W1_root, W2_rel, W2_root,
    b2.reshape(1, 128))

  (s2,) = _sc_agg_128(p2, srcp, dstp, z128)

  p3, r3 = pl.pallas_call(
      _tc2_body,
      grid=(_GRID,),
      in_specs=[
          _part_blk(128),
          _row_blk((128,)),
          _row_blk((8,)),
          _full_blk((128, 2)),
          _full_blk((128, 2)),
          _full_blk((128, 2)),
          _full_blk((128, 2)),
          _full_blk((1, 2)),
          _full_blk((1, 2)),
      ],
      out_specs=[_row_blk((16,)), _row_blk((16,))],
      out_shape=[
          jax.ShapeDtypeStruct((N_NODES, 16), jnp.float32),
          jax.ShapeDtypeStruct((N_NODES, 16), jnp.float32),
      ],
  )(s2, r2, ic, Wmu_rel, Wls_rel, Wmu_root, Wls_root,
    bmu.reshape(1, 2), bls.reshape(1, 2))

  (s3,) = _sc_agg_16(p3, srcp, dstp, z16)

  out = (s3[0] + s3[1]) * ic[:, :1] + r3

  return out[:, 0:2], out[:, 2:4]
